# speedup vs baseline: 18.5034x; 18.5034x over previous
import functools

import jax
import jax.numpy as jnp
from jax.experimental import pallas as pl
from jax.experimental.pallas import tpu as pltpu

TOPK = 32
ROWS_PER_BLOCK = 256
ROW_LEN = 2048


def _mask_kernel(x_ref, o_ref):
    x = x_ref[...]
    b = jax.lax.bitcast_convert_type(x, jnp.uint32)
    sign = b >> 31
    flip = jnp.where(sign == 1, jnp.uint32(0xFFFFFFFF), jnp.uint32(0x80000000))
    key = b ^ flip

    t = jnp.zeros((x.shape[0], 1), dtype=jnp.uint32)
    for bit in range(31, -1, -1):
        cand = t | jnp.uint32(1 << bit)
        cnt = jnp.sum(jnp.where(key >= cand, 1, 0).astype(jnp.float32),
                      axis=1, keepdims=True)
        t = jnp.where(cnt >= float(TOPK), cand, t)
    o_ref[...] = jnp.where(key >= t, x, jnp.float32(0.0))


@jax.jit
def kernel(T):
    shape = T.shape
    L = shape[-1]
    flat = T.reshape(-1, L)
    n_rows = flat.shape[0]
    grid = n_rows // ROWS_PER_BLOCK
    out = pl.pallas_call(
        _mask_kernel,
        grid=(grid,),
        in_specs=[pl.BlockSpec((ROWS_PER_BLOCK, L), lambda i: (i, 0))],
        out_specs=pl.BlockSpec((ROWS_PER_BLOCK, L), lambda i: (i, 0)),
        out_shape=jax.ShapeDtypeStruct((n_rows, L), jnp.float32),
    )(flat)
    return out.reshape(shape)
